# Initial kernel scaffold; baseline (speedup 1.0000x reference)
import jax
import jax.numpy as jnp
from jax.experimental import pallas as pl


def kernel(pos, x, ws1, bs1, w11, b11, w2a1, b2a1, w2b1, b2b1, ws2, bs2, w12, b12, w2a2, b2a2, w2b2, b2b2, w3, b3):
    raise NotImplementedError("write your pallas kernel here")


# speedup vs baseline: 2.2816x; 2.2816x over previous
import jax
import jax.numpy as jnp
from jax.experimental import pallas as pl
from jax.experimental.pallas import tpu as pltpu

_NEG_SLOPE = 0.2


def _lrelu(v):
    return jnp.maximum(v, _NEG_SLOPE * v)


def _round_up(n, m):
    return ((n + m - 1) // m) * m


def _fused_kernel(pos_ref, x_ref, wc1_ref, w2a1_ref, b2a1_ref, w2b1_ref,
                  b2b1_ref, wcf_ref, bc2_ref, wc2p_ref, wcx_ref, w2a2_ref,
                  b2a2_ref, w2b2_ref, b2b2_ref, w3_ref, b3_ref, o_ref):
    f32 = jnp.float32
    dn = (((1,), (0,)), ((), ()))

    p = pos_ref[0]
    ones = jnp.ones((1, p.shape[1]), f32)
    p_aug = jnp.concatenate([p, ones], axis=0)

    xc1 = jax.lax.dot_general(wc1_ref[...], p_aug, dn,
                              preferred_element_type=f32)
    s1 = xc1[0:128, :]
    h1 = _lrelu(xc1[128:192, :])
    h2 = _lrelu(jnp.dot(w2a1_ref[...], h1, preferred_element_type=f32)
                + b2a1_ref[...])
    y1 = (jnp.dot(w2b1_ref[...], h2, preferred_element_type=f32)
          + b2b1_ref[...] + s1)

    pooled = jnp.max(y1, axis=1, keepdims=True)
    pterm = (jax.lax.dot_general(wcf_ref[...], pooled, dn,
                                 preferred_element_type=f32)
             + bc2_ref[...])

    wc2_aug = jnp.concatenate([wc2p_ref[...], pterm], axis=1)
    xc2 = (jax.lax.dot_general(wc2_aug, p_aug, dn,
                               preferred_element_type=f32)
           + jnp.dot(wcx_ref[...], x_ref[0], preferred_element_type=f32))
    s2 = xc2[0:64, :]
    h = _lrelu(xc2[64:192, :])
    g2 = _lrelu(jnp.dot(w2a2_ref[...], h, preferred_element_type=f32)
                + b2a2_ref[...])
    y2 = (jnp.dot(w2b2_ref[...], g2, preferred_element_type=f32)
          + b2b2_ref[...] + s2)
    feat = _lrelu(y2)
    out = (jnp.dot(w3_ref[...], feat, preferred_element_type=f32)
           + b3_ref[...])
    o_ref[0] = out.astype(o_ref.dtype)


def kernel(pos, x, ws1, bs1, w11, b11, w2a1, b2a1, w2b1, b2b1, ws2, bs2,
           w12, b12, w2a2, b2a2, w2b2, b2b2, w3, b3):
    f32 = jnp.float32
    B, cp, N = pos.shape
    cx = x.shape[1]
    c1_out = ws1.shape[1]
    c2_out = ws2.shape[1]
    c3_out = w3.shape[1]
    fused1 = c1_out + w11.shape[1]
    fused2 = c2_out + w12.shape[1]

    n_pad = _round_up(N, 128)
    pad = n_pad - N
    if pad:
        pos = jnp.pad(pos, ((0, 0), (0, 0), (0, pad)), mode="edge")
        x = jnp.pad(x, ((0, 0), (0, 0), (0, pad)), mode="edge")

    wc1 = jnp.concatenate(
        [jnp.concatenate([ws1.T, w11.T], axis=0),
         jnp.concatenate([bs1, b11])[:, None]], axis=1).astype(f32)
    b2a1c = b2a1.astype(f32)[:, None]
    b2b1c = b2b1.astype(f32)[:, None]
    w2a1T = w2a1.T.astype(f32)
    w2b1T = w2b1.T.astype(f32)

    wc2T = jnp.concatenate([ws2.T, w12.T], axis=0).astype(f32)
    wc2p = wc2T[:, :cp]
    wcf = wc2T[:, cp:cp + c1_out]
    wcx = wc2T[:, cp + c1_out:]
    bc2 = jnp.concatenate([bs2, b12]).astype(f32)[:, None]
    w2a2T = w2a2.T.astype(f32)
    w2b2T = w2b2.T.astype(f32)
    b2a2c = b2a2.astype(f32)[:, None]
    b2b2c = b2b2.astype(f32)[:, None]
    w3T = w3.T.astype(f32)
    b3c = b3.astype(f32)[:, None]

    flops = 2 * B * n_pad * (cp * fused1 + 64 * c1_out + c1_out * c1_out
                             + (cp + cx) * fused2 + 128 * c2_out
                             + c2_out * c2_out + c2_out * c3_out)
    bytes_accessed = int(4 * (pos.size + x.size + B * c3_out * n_pad
                              + fused1 * 4 + c1_out * 64 + c1_out * c1_out
                              + fused2 * (4 + c1_out + cx)
                              + 128 * c2_out + c2_out * c2_out
                              + c3_out * c2_out))

    full = lambda *s: pl.BlockSpec(s, lambda b: (0,) * len(s))
    out_pad = pl.pallas_call(
        _fused_kernel,
        out_shape=jax.ShapeDtypeStruct((B, c3_out, n_pad), pos.dtype),
        grid=(B,),
        in_specs=[
            pl.BlockSpec((1, cp, n_pad), lambda b: (b, 0, 0)),
            pl.BlockSpec((1, cx, n_pad), lambda b: (b, 0, 0)),
            full(fused1, cp + 1),
            full(c1_out, 64), full(c1_out, 1),
            full(c1_out, c1_out), full(c1_out, 1),
            full(fused1, c1_out), full(fused2, 1),
            full(fused2, cp), full(fused2, cx),
            full(c2_out, 128), full(c2_out, 1),
            full(c2_out, c2_out), full(c2_out, 1),
            full(c3_out, c2_out), full(c3_out, 1),
        ],
        out_specs=pl.BlockSpec((1, c3_out, n_pad), lambda b: (b, 0, 0)),
        compiler_params=pltpu.CompilerParams(
            dimension_semantics=("parallel",),
            vmem_limit_bytes=48 * 1024 * 1024),
        cost_estimate=pl.CostEstimate(flops=flops, transcendentals=0,
                                      bytes_accessed=bytes_accessed),
    )(pos.astype(f32), x.astype(f32), wc1, w2a1T, b2a1c, w2b1T, b2b1c,
      wcf, bc2, wc2p, wcx, w2a2T, b2a2c, w2b2T, b2b2c, w3T, b3c)

    return out_pad[:, :, :N] if pad else out_pad


# speedup vs baseline: 2.5963x; 1.1379x over previous
import jax
import jax.numpy as jnp
from jax.experimental import pallas as pl
from jax.experimental.pallas import tpu as pltpu

_NEG_SLOPE = 0.2


def _lrelu(v):
    return jnp.maximum(v, _NEG_SLOPE * v)


def _round_up(n, m):
    return ((n + m - 1) // m) * m


def _fused_kernel(pos_ref, x_ref, w11a_ref, w2a1_ref, b2a1_ref, wy1_ref,
                  wcf_ref, bc2_ref, wc2p_ref, wcx_ref, w2a2_ref, b2a2_ref,
                  w2b2_ref, b2b2_ref, w3_ref, b3_ref, o_ref, s_ref):
    f32 = jnp.float32
    dn = (((1,), (0,)), ((), ()))
    T = pos_ref.shape[2]

    p = pos_ref[0]
    aug = jnp.concatenate(
        [p, jnp.ones((1, T), f32), jnp.zeros((4, T), f32)], axis=0)
    s_ref[128:136, :] = aug
    s_ref[136:264, :] = x_ref[0]

    h1 = _lrelu(jax.lax.dot_general(w11a_ref[...], s_ref[128:136, :], dn,
                                    preferred_element_type=f32))
    h2 = _lrelu(jnp.dot(w2a1_ref[...], h1, preferred_element_type=f32)
                + b2a1_ref[...])
    s_ref[0:128, :] = h2
    y1 = jax.lax.dot_general(wy1_ref[...], s_ref[0:136, :], dn,
                             preferred_element_type=f32)

    pooled = jnp.max(y1, axis=1, keepdims=True)
    pterm = (jax.lax.dot_general(wcf_ref[...], pooled, dn,
                                 preferred_element_type=f32)
             + bc2_ref[...])

    wxc2 = jnp.concatenate(
        [wc2p_ref[...], pterm, jnp.zeros((192, 4), f32), wcx_ref[...]],
        axis=1)
    xc2 = jax.lax.dot_general(wxc2, s_ref[128:264, :], dn,
                              preferred_element_type=f32)
    h = _lrelu(xc2[64:192, :])
    g2 = _lrelu(jnp.dot(w2a2_ref[...], h, preferred_element_type=f32)
                + b2a2_ref[...])
    s_ref[264:328, :] = g2
    wy2 = jnp.concatenate(
        [wc2p_ref[0:64, :], pterm[0:64, :] + b2b2_ref[...],
         jnp.zeros((64, 4), f32), wcx_ref[0:64, :], w2b2_ref[...]],
        axis=1)
    y2 = jax.lax.dot_general(wy2, s_ref[128:328, :], dn,
                             preferred_element_type=f32)
    feat = _lrelu(y2)
    out = (jnp.dot(w3_ref[...], feat, preferred_element_type=f32)
           + b3_ref[...])
    o_ref[0] = out.astype(o_ref.dtype)


def kernel(pos, x, ws1, bs1, w11, b11, w2a1, b2a1, w2b1, b2b1, ws2, bs2,
           w12, b12, w2a2, b2a2, w2b2, b2b2, w3, b3):
    f32 = jnp.float32
    B, cp, N = pos.shape
    cx = x.shape[1]
    c1_out = ws1.shape[1]
    c2_out = ws2.shape[1]
    c3_out = w3.shape[1]
    fused2 = c2_out + w12.shape[1]

    n_pad = _round_up(N, 128)
    pad = n_pad - N
    if pad:
        pos = jnp.pad(pos, ((0, 0), (0, 0), (0, pad)), mode="edge")
        x = jnp.pad(x, ((0, 0), (0, 0), (0, pad)), mode="edge")

    w11a = jnp.concatenate(
        [w11.T, b11[:, None], jnp.zeros((64, 4), f32)], axis=1).astype(f32)
    w2a1T = w2a1.T.astype(f32)
    b2a1c = b2a1.astype(f32)[:, None]
    wy1 = jnp.concatenate(
        [w2b1.T, ws1.T, (bs1 + b2b1)[:, None], jnp.zeros((128, 4), f32)],
        axis=1).astype(f32)

    wc2T = jnp.concatenate([ws2.T, w12.T], axis=0).astype(f32)
    wc2p = wc2T[:, :cp]
    wcf = wc2T[:, cp:cp + c1_out]
    wcx = wc2T[:, cp + c1_out:]
    bc2 = jnp.concatenate([bs2, b12]).astype(f32)[:, None]
    w2a2T = w2a2.T.astype(f32)
    b2a2c = b2a2.astype(f32)[:, None]
    w2b2T = w2b2.T.astype(f32)
    b2b2c = b2b2.astype(f32)[:, None]
    w3T = w3.T.astype(f32)
    b3c = b3.astype(f32)[:, None]

    flops = 2 * B * n_pad * (8 * 64 + 64 * c1_out + 136 * c1_out
                             + 136 * fused2 + 128 * c2_out
                             + 200 * c2_out + c2_out * c3_out)
    bytes_accessed = int(4 * (pos.size + x.size + B * c3_out * n_pad))

    full = lambda *s: pl.BlockSpec(s, lambda b: (0,) * len(s))
    out_pad = pl.pallas_call(
        _fused_kernel,
        out_shape=jax.ShapeDtypeStruct((B, c3_out, n_pad), pos.dtype),
        grid=(B,),
        in_specs=[
            pl.BlockSpec((1, cp, n_pad), lambda b: (b, 0, 0)),
            pl.BlockSpec((1, cx, n_pad), lambda b: (b, 0, 0)),
            full(64, 8),
            full(c1_out, 64), full(c1_out, 1),
            full(c1_out, 136),
            full(fused2, c1_out), full(fused2, 1),
            full(fused2, cp), full(fused2, cx),
            full(c2_out, 128), full(c2_out, 1),
            full(c2_out, c2_out), full(c2_out, 1),
            full(c3_out, c2_out), full(c3_out, 1),
        ],
        out_specs=pl.BlockSpec((1, c3_out, n_pad), lambda b: (b, 0, 0)),
        scratch_shapes=[pltpu.VMEM((328, n_pad), f32)],
        compiler_params=pltpu.CompilerParams(
            dimension_semantics=("parallel",),
            vmem_limit_bytes=48 * 1024 * 1024),
        cost_estimate=pl.CostEstimate(flops=flops, transcendentals=0,
                                      bytes_accessed=bytes_accessed),
    )(pos.astype(f32), x.astype(f32), w11a, w2a1T, b2a1c, wy1,
      wcf, bc2, wc2p, wcx, w2a2T, b2a2c, w2b2T, b2b2c, w3T, b3c)

    return out_pad[:, :, :N] if pad else out_pad
